# Initial kernel scaffold; baseline (speedup 1.0000x reference)
import jax
import jax.numpy as jnp
from jax.experimental import pallas as pl


def kernel(x, edge_index, edge_attr, W, att_src, att_dst, W_edge, att_edge, bias, mid_weight):
    raise NotImplementedError("write your pallas kernel here")


# speedup vs baseline: 16.2816x; 16.2816x over previous
import functools

import jax
import jax.numpy as jnp
import numpy as np
from jax import lax
from jax.experimental import pallas as pl
from jax.experimental.pallas import tpu as pltpu
from jax.experimental.pallas import tpu_sc as plsc

N = 10000
E = 320000
D = 128
ED = 16
ALPHA = 0.2
THETA = 0.2

NC = 2
NS = 16
NW = NC * NS
EPC = E // NC
EPT = E // NW
CH = 64
NCHUNK = EPT // CH
TAIL = EPT - NCHUNK * CH


def _zero_slabs(si, zsrc, sh_ref):
    def _zfill(start, count):
        full, rem = divmod(count, CH)
        for k in range(full):
            pltpu.sync_copy(zsrc, sh_ref.at[pl.ds(start + k * CH, CH), :])
        if rem:
            pltpu.sync_copy(zsrc.at[pl.ds(0, rem), :],
                            sh_ref.at[pl.ds(start + full * CH, rem), :])

    @pl.when(si < NS - 1)
    def _():
        _zfill(si * 640, 640)

    @pl.when(si == NS - 1)
    def _():
        _zfill(9600, 400)


def _writeback_slabs(si, sh_ref, hbm_slab):
    @pl.when(si < NS - 1)
    def _():
        pltpu.sync_copy(sh_ref.at[pl.ds(si * 640, 640), :],
                        hbm_slab.at[pl.ds(si * 640, 640), :])

    @pl.when(si == NS - 1)
    def _():
        pltpu.sync_copy(sh_ref.at[pl.ds(9600, 400), :],
                        hbm_slab.at[pl.ds(9600, 400), :])


@functools.cache
def _mesh():
    return plsc.VectorSubcoreMesh(core_axis_name="c", subcore_axis_name="s",
                                  num_cores=NC, num_subcores=NS)


def _sc_pre_body(dst_hbm, ea_hbm, lsd_hbm, dstv, earows, ldrows, dstv_t,
                 lsd_sh):
    ci = lax.axis_index("c")
    si = lax.axis_index("s")
    z16 = jnp.zeros((16,), jnp.float32)
    onehot = jnp.where(lax.iota(jnp.int32, 16) == 0, 1.0, 0.0)

    def _zrow(r, _):
        for k in range(8):
            ldrows[r, pl.ds(k * 16, 16)] = z16
        return _
    lax.fori_loop(0, CH, _zrow, None)
    _zero_slabs(si, ldrows, lsd_sh)

    def _onerow(r, _):
        ldrows[r, pl.ds(16, 16)] = onehot
        return _
    lax.fori_loop(0, CH, _onerow, None)
    plsc.subcore_barrier()

    def _chunk(base, K, dv):
        pltpu.sync_copy(dst_hbm.at[pl.ds(base, K)], dv)
        pltpu.sync_copy(ea_hbm.at[pl.ds(base, K), :],
                        earows.at[pl.ds(0, K), :])
        for e in range(K):
            ldrows[e, 0:16] = earows[e, :]
        pltpu.sync_copy(ldrows.at[pl.ds(0, K), :], lsd_sh.at[dv], add=True)

    tile_base = ci * EPC + si * EPT

    def _loop(c, _):
        _chunk(tile_base + c * CH, CH, dstv)
        return _
    lax.fori_loop(0, NCHUNK, _loop, None)
    if TAIL:
        _chunk(tile_base + NCHUNK * CH, TAIL, dstv_t)

    plsc.subcore_barrier()
    _writeback_slabs(si, lsd_sh, lsd_hbm.at[ci])


@functools.cache
def _sc_pre():
    return pl.kernel(
        _sc_pre_body,
        out_type=jax.ShapeDtypeStruct((NC, N, D), jnp.float32),
        mesh=_mesh(),
        compiler_params=pltpu.CompilerParams(needs_layout_passes=False),
        scratch_types=[
            pltpu.VMEM((CH,), jnp.int32),
            pltpu.VMEM((CH, ED), jnp.float32),
            pltpu.VMEM((CH, D), jnp.float32),
            pltpu.VMEM((TAIL,), jnp.int32),
            pltpu.VMEM_SHARED((N, D), jnp.float32),
        ],
    )


def _sc_edge_body(layer, src_hbm, dst_hbm, ae_hbm, h_hbm, as2d_hbm,
                  p_hbm, dw_hbm,
                  as2v, srcv, dstv, aev, rows, den, srcv_t, dstv_t,
                  p_sh):
    ci = lax.axis_index("c")
    si = lax.axis_index("s")
    z16 = jnp.zeros((16,), jnp.float32)

    for q in range(4):
        pltpu.sync_copy(as2d_hbm.at[pl.ds(q * (N // 2), N // 2)],
                        as2v.at[pl.ds(q * (N // 2), N // 2)])

    def _zrow(r, _):
        for k in range(8):
            rows[r, pl.ds(k * 16, 16)] = z16
        return _
    lax.fori_loop(0, CH, _zrow, None)

    def _zden(r, _):
        den[pl.ds(r * 16, 16)] = z16
        return _
    lax.fori_loop(0, N // 16, _zden, None)
    _zero_slabs(si, rows, p_sh)
    plsc.subcore_barrier()

    def _chunk(base, K, sv, dv):
        pltpu.sync_copy(src_hbm.at[pl.ds(base, K)], sv)
        pltpu.sync_copy(dst_hbm.at[pl.ds(base, K)], dv)
        pltpu.sync_copy(ae_hbm.at[pl.ds(layer * E + base, K)],
                        aev.at[pl.ds(0, K)])
        pltpu.sync_copy(h_hbm.at[sv], rows.at[pl.ds(0, K), :])
        for g in range(K // 16):
            sidx = sv[pl.ds(g * 16, 16)]
            didx = dv[pl.ds(g * 16, 16)]
            a = (plsc.load_gather(as2v, [sidx * 2])
                 + plsc.load_gather(as2v, [didx * 2 + 1])
                 + aev[pl.ds(g * 16, 16)])
            a = jnp.where(a >= 0, a, 0.2 * a)
            w = jnp.exp(a)
            plsc.addupdate_scatter(den, [didx], w)
            for l in range(16):
                c = w[l]
                e = g * 16 + l
                for k in range(8):
                    rows[e, pl.ds(k * 16, 16)] = rows[e, pl.ds(k * 16, 16)] * c
        pltpu.sync_copy(rows.at[pl.ds(0, K), :], p_sh.at[dv], add=True)

    tile_base = ci * EPC + si * EPT

    def _loop(c, _):
        _chunk(tile_base + c * CH, CH, srcv, dstv)
        return _
    lax.fori_loop(0, NCHUNK, _loop, None)
    if TAIL:
        _chunk(tile_base + NCHUNK * CH, TAIL, srcv_t, dstv_t)

    pltpu.sync_copy(den, dw_hbm.at[pl.ds((ci * NS + si) * N, N)])
    plsc.subcore_barrier()
    _writeback_slabs(si, p_sh, p_hbm.at[ci])


@functools.cache
def _make_sc_edge(layer):
    return pl.kernel(
        functools.partial(_sc_edge_body, layer),
        out_type=(jax.ShapeDtypeStruct((NC, N, D), jnp.float32),
                  jax.ShapeDtypeStruct((NW * N,), jnp.float32)),
        mesh=_mesh(),
        compiler_params=pltpu.CompilerParams(needs_layout_passes=False),
        scratch_types=[
            pltpu.VMEM((2 * N,), jnp.float32),
            pltpu.VMEM((CH,), jnp.int32),
            pltpu.VMEM((CH,), jnp.int32),
            pltpu.VMEM((CH,), jnp.float32),
            pltpu.VMEM((CH, D), jnp.float32),
            pltpu.VMEM((N,), jnp.float32),
            pltpu.VMEM((TAIL,), jnp.int32),
            pltpu.VMEM((TAIL,), jnp.int32),
            pltpu.VMEM_SHARED((N, D), jnp.float32),
        ],
    )


def _sc_edge(i):
    return _make_sc_edge(i)


RPT = 320


def _sc_den_body(dwp_hbm, dwr_hbm, acc, tmp, exprows):
    ci = lax.axis_index("c")
    si = lax.axis_index("s")
    w = si * NC + ci
    z16 = jnp.zeros((16,), jnp.float32)
    onehot = jnp.where(lax.iota(jnp.int32, 16) == 0, 1.0, 0.0)

    def _run(row0, nrows):
        def _z(r, _):
            acc[pl.ds(r * 16, 16)] = z16
            return _
        lax.fori_loop(0, nrows // 16, _z, None)
        for w2 in range(NW):
            pltpu.sync_copy(dwp_hbm.at[pl.ds(w2 * N + row0, nrows)],
                            tmp.at[pl.ds(0, nrows)])

            def _add(r, _):
                sl = pl.ds(r * 16, 16)
                acc[sl] = acc[sl] + tmp[sl]
                return _
            lax.fori_loop(0, nrows // 16, _add, None)

        full, rem = divmod(nrows, CH)
        for j in range(full):
            def _expj(g, _):
                v = acc[pl.ds(j * CH + g * 16, 16)]
                for l in range(16):
                    exprows[g * 16 + l, :] = v[l] * onehot
                return _
            lax.fori_loop(0, CH // 16, _expj, None)
            pltpu.sync_copy(exprows,
                            dwr_hbm.at[pl.ds(row0 + j * CH, CH), :])
        if rem:
            def _expr(g, _):
                v = acc[pl.ds(full * CH + g * 16, 16)]
                for l in range(16):
                    exprows[g * 16 + l, :] = v[l] * onehot
                return _
            lax.fori_loop(0, rem // 16, _expr, None)
            pltpu.sync_copy(exprows.at[pl.ds(0, rem), :],
                            dwr_hbm.at[pl.ds(row0 + full * CH, rem), :])

    @pl.when(w < NW - 1)
    def _():
        _run(w * RPT, RPT)

    @pl.when(w == NW - 1)
    def _():
        _run((NW - 1) * RPT, N - (NW - 1) * RPT)


@functools.cache
def _sc_den():
    return pl.kernel(
        _sc_den_body,
        out_type=jax.ShapeDtypeStruct((N, 16), jnp.float32),
        mesh=_mesh(),
        compiler_params=pltpu.CompilerParams(needs_layout_passes=False),
        scratch_types=[
            pltpu.VMEM((RPT,), jnp.float32),
            pltpu.VMEM((RPT,), jnp.float32),
            pltpu.VMEM((CH, 16), jnp.float32),
        ],
    )


_HI = jax.lax.Precision.HIGHEST
NB = 2000
NG = N // NB
EB = 2560
EG = E // EB


def _edge_attn_vectors(we_ref, aee_ref):
    cols = [lax.dot_general(we_ref[l], aee_ref[l][:, None],
                            (((1,), (0,)), ((), ())), precision=_HI)
            for l in range(5)]
    return jnp.concatenate(cols, axis=1)


def _prep_edges_body(ea_ref, we_ref, aee_ref, out_ref):
    V = _edge_attn_vectors(we_ref, aee_ref)
    out_ref[...] = lax.dot_general(V, ea_ref[...], (((0,), (1,)), ((), ())),
                                   precision=_HI)


_prep_edges = pl.pallas_call(
    _prep_edges_body,
    grid=(EG,),
    in_specs=[
        pl.BlockSpec((EB, ED), lambda j: (j, 0)),
        pl.BlockSpec((5, ED, D), lambda j: (0, 0, 0)),
        pl.BlockSpec((5, D), lambda j: (0, 0)),
    ],
    out_specs=pl.BlockSpec((5, EB), lambda j: (0, j)),
    out_shape=jax.ShapeDtypeStruct((5, E), jnp.float32),
)


def _prep_loop_body(lsd_ref, we_ref, aee_ref, out_ref):
    s = lsd_ref[0] + lsd_ref[1]
    la = s[:, 0:16] / jnp.maximum(s[:, 16:17], 1.0)
    V = _edge_attn_vectors(we_ref, aee_ref)
    out_ref[...] = lax.dot_general(la, V, (((1,), (0,)), ((), ())),
                                   precision=_HI)


_prep_loop = pl.pallas_call(
    _prep_loop_body,
    grid=(NG,),
    in_specs=[
        pl.BlockSpec((NC, NB, D), lambda j: (0, j, 0)),
        pl.BlockSpec((5, ED, D), lambda j: (0, 0, 0)),
        pl.BlockSpec((5, D), lambda j: (0, 0)),
    ],
    out_specs=pl.BlockSpec((NB, 5), lambda j: (j, 0)),
    out_shape=jax.ShapeDtypeStruct((N, 5), jnp.float32),
)


def _project(xc, w_ref, a2_ref):
    h = jnp.dot(xc, w_ref[...], precision=_HI)
    as2d = jnp.dot(h, a2_ref[...], precision=_HI)
    return h, as2d


def _pre0_body(x_ref, w_ref, a2_ref, h_ref, as2d_ref):
    h, as2d = _project(x_ref[...], w_ref, a2_ref)
    h_ref[...] = h
    as2d_ref[...] = as2d


_pre0 = pl.pallas_call(
    _pre0_body,
    grid=(NG,),
    in_specs=[
        pl.BlockSpec((NB, D), lambda j: (j, 0)),
        pl.BlockSpec((D, D), lambda j: (0, 0)),
        pl.BlockSpec((D, 2), lambda j: (0, 0)),
    ],
    out_specs=[
        pl.BlockSpec((NB, D), lambda j: (j, 0)),
        pl.BlockSpec((NB, 2), lambda j: (j, 0)),
    ],
    out_shape=[
        jax.ShapeDtypeStruct((N, D), jnp.float32),
        jax.ShapeDtypeStruct((N, 2), jnp.float32),
    ],
)


def _normalize(layer, p_ref, dw_ref, h_ref, as2d_ref, ael_ref, b_ref):
    p = p_ref[0] + p_ref[1]
    den = dw_ref[:, 0:1]
    al = as2d_ref[:, 0:1] + as2d_ref[:, 1:2] + ael_ref[:, layer:layer + 1]
    al = jnp.where(al >= 0, al, 0.2 * al)
    wl = jnp.exp(al)
    return (p + wl * h_ref[...]) / (den + wl + 1e-16) + b_ref[...]


def _post0_body(p_ref, dw_ref, h_ref, as2d_ref, ael_ref, b_ref, w_ref, a2_ref,
                x0_ref, h1_ref, as2d1_ref):
    x0 = _normalize(0, p_ref, dw_ref, h_ref, as2d_ref, ael_ref, b_ref)
    x0_ref[...] = x0
    h, as2d = _project(x0, w_ref, a2_ref)
    h1_ref[...] = h
    as2d1_ref[...] = as2d


_post0 = pl.pallas_call(
    _post0_body,
    grid=(NG,),
    in_specs=[
        pl.BlockSpec((NC, NB, D), lambda j: (0, j, 0)),
        pl.BlockSpec((NB, 16), lambda j: (j, 0)),
        pl.BlockSpec((NB, D), lambda j: (j, 0)),
        pl.BlockSpec((NB, 2), lambda j: (j, 0)),
        pl.BlockSpec((NB, 5), lambda j: (j, 0)),
        pl.BlockSpec((1, D), lambda j: (0, 0)),
        pl.BlockSpec((D, D), lambda j: (0, 0)),
        pl.BlockSpec((D, 2), lambda j: (0, 0)),
    ],
    out_specs=[
        pl.BlockSpec((NB, D), lambda j: (j, 0)),
        pl.BlockSpec((NB, D), lambda j: (j, 0)),
        pl.BlockSpec((NB, 2), lambda j: (j, 0)),
    ],
    out_shape=[
        jax.ShapeDtypeStruct((N, D), jnp.float32),
        jax.ShapeDtypeStruct((N, D), jnp.float32),
        jax.ShapeDtypeStruct((N, 2), jnp.float32),
    ],
)


def _post_mid_body(layer, p_ref, dw_ref, h_ref, as2d_ref, ael_ref, b_ref,
                   x0_ref, mid_ref, w_ref, a2_ref,
                   h1_ref, as2d1_ref):
    cx = _normalize(layer, p_ref, dw_ref, h_ref, as2d_ref, ael_ref, b_ref)
    beta = float(np.log(THETA / layer + 1.0))
    xc = (1.0 - ALPHA) * cx + ALPHA * x0_ref[...]
    xm = jnp.dot(xc, mid_ref[...], precision=_HI)
    xc = (1.0 - beta) * xc + beta * xm
    xc = jnp.where(xc >= 0, xc, 0.01 * xc)
    h, as2d = _project(xc, w_ref, a2_ref)
    h1_ref[...] = h
    as2d1_ref[...] = as2d


def _make_post_mid(layer):
    return pl.pallas_call(
        functools.partial(_post_mid_body, layer),
        grid=(NG,),
        in_specs=[
            pl.BlockSpec((NC, NB, D), lambda j: (0, j, 0)),
            pl.BlockSpec((NB, 16), lambda j: (j, 0)),
            pl.BlockSpec((NB, D), lambda j: (j, 0)),
            pl.BlockSpec((NB, 2), lambda j: (j, 0)),
            pl.BlockSpec((NB, 5), lambda j: (j, 0)),
            pl.BlockSpec((1, D), lambda j: (0, 0)),
            pl.BlockSpec((NB, D), lambda j: (j, 0)),
            pl.BlockSpec((D, D), lambda j: (0, 0)),
            pl.BlockSpec((D, D), lambda j: (0, 0)),
            pl.BlockSpec((D, 2), lambda j: (0, 0)),
        ],
        out_specs=[
            pl.BlockSpec((NB, D), lambda j: (j, 0)),
            pl.BlockSpec((NB, 2), lambda j: (j, 0)),
        ],
        out_shape=[
            jax.ShapeDtypeStruct((N, D), jnp.float32),
            jax.ShapeDtypeStruct((N, 2), jnp.float32),
        ],
    )


_post_mid = {i: _make_post_mid(i) for i in (1, 2, 3)}


def _post_final_body(p_ref, dw_ref, h_ref, as2d_ref, ael_ref, b_ref, out_ref):
    out_ref[...] = _normalize(4, p_ref, dw_ref, h_ref, as2d_ref, ael_ref,
                              b_ref)


_post_final = pl.pallas_call(
    _post_final_body,
    grid=(NG,),
    in_specs=[
        pl.BlockSpec((NC, NB, D), lambda j: (0, j, 0)),
        pl.BlockSpec((NB, 16), lambda j: (j, 0)),
        pl.BlockSpec((NB, D), lambda j: (j, 0)),
        pl.BlockSpec((NB, 2), lambda j: (j, 0)),
        pl.BlockSpec((NB, 5), lambda j: (j, 0)),
        pl.BlockSpec((1, D), lambda j: (0, 0)),
    ],
    out_specs=pl.BlockSpec((NB, D), lambda j: (j, 0)),
    out_shape=jax.ShapeDtypeStruct((N, D), jnp.float32),
)


def kernel(x, edge_index, edge_attr, W, att_src, att_dst, W_edge, att_edge,
           bias, mid_weight):
    src = edge_index[0]
    dst = edge_index[1]
    a2 = [jnp.stack([att_src[i], att_dst[i]], axis=1) for i in range(5)]
    b = [bias[i][None, :] for i in range(5)]

    lsd = _sc_pre()(dst, edge_attr)
    ael = _prep_loop(lsd, W_edge, att_edge)
    ae = _prep_edges(edge_attr, W_edge, att_edge).reshape(5 * E)

    h, as2d = _pre0(x, W[0], a2[0])
    P, Dwp = _sc_edge(0)(src, dst, ae, h, as2d.reshape(2 * N))
    Dw = _sc_den()(Dwp)
    x0, h, as2d = _post0(P, Dw, h, as2d, ael, b[0], W[1], a2[1])
    for i in (1, 2, 3):
        P, Dwp = _sc_edge(i)(src, dst, ae, h, as2d.reshape(2 * N))
        Dw = _sc_den()(Dwp)
        h, as2d = _post_mid[i](P, Dw, h, as2d, ael, b[i], x0,
                               mid_weight[i - 1], W[i + 1], a2[i + 1])
    P, Dwp = _sc_edge(4)(src, dst, ae, h, as2d.reshape(2 * N))
    Dw = _sc_den()(Dwp)
    return _post_final(P, Dw, h, as2d, ael, b[4])


# speedup vs baseline: 19.4439x; 1.1942x over previous
import functools

import jax
import jax.numpy as jnp
import numpy as np
from jax import lax
from jax.experimental import pallas as pl
from jax.experimental.pallas import tpu as pltpu
from jax.experimental.pallas import tpu_sc as plsc

N = 10000
E = 320000
D = 128
ED = 16
ALPHA = 0.2
THETA = 0.2

NC = 2
NS = 16
NW = NC * NS
EPC = E // NC
EPT = E // NW
CH = 96
NCHUNK = EPT // CH
TAIL = EPT - NCHUNK * CH


def _zero_slabs(si, zsrc, sh_ref):
    def _zfill(start, count):
        full, rem = divmod(count, CH)
        for k in range(full):
            pltpu.sync_copy(zsrc, sh_ref.at[pl.ds(start + k * CH, CH), :])
        if rem:
            pltpu.sync_copy(zsrc.at[pl.ds(0, rem), :],
                            sh_ref.at[pl.ds(start + full * CH, rem), :])

    @pl.when(si < NS - 1)
    def _():
        _zfill(si * 640, 640)

    @pl.when(si == NS - 1)
    def _():
        _zfill(9600, 400)


def _writeback_slabs(si, sh_ref, hbm_slab):
    @pl.when(si < NS - 1)
    def _():
        pltpu.sync_copy(sh_ref.at[pl.ds(si * 640, 640), :],
                        hbm_slab.at[pl.ds(si * 640, 640), :])

    @pl.when(si == NS - 1)
    def _():
        pltpu.sync_copy(sh_ref.at[pl.ds(9600, 400), :],
                        hbm_slab.at[pl.ds(9600, 400), :])


@functools.cache
def _mesh():
    return plsc.VectorSubcoreMesh(core_axis_name="c", subcore_axis_name="s",
                                  num_cores=NC, num_subcores=NS)


def _sc_pre_body(dst_hbm, ea_hbm, lsd_hbm, dstv, earows, ldrows, dstv_t,
                 lsd_sh):
    ci = lax.axis_index("c")
    si = lax.axis_index("s")
    z16 = jnp.zeros((16,), jnp.float32)
    onehot = jnp.where(lax.iota(jnp.int32, 16) == 0, 1.0, 0.0)

    def _zrow(r, _):
        for k in range(8):
            ldrows[r, pl.ds(k * 16, 16)] = z16
        return _
    lax.fori_loop(0, CH, _zrow, None)
    _zero_slabs(si, ldrows, lsd_sh)

    def _onerow(r, _):
        ldrows[r, pl.ds(16, 16)] = onehot
        return _
    lax.fori_loop(0, CH, _onerow, None)
    plsc.subcore_barrier()

    def _chunk(base, K, dv):
        pltpu.sync_copy(dst_hbm.at[pl.ds(base, K)], dv)
        pltpu.sync_copy(ea_hbm.at[pl.ds(base, K), :],
                        earows.at[pl.ds(0, K), :])
        for e in range(K):
            ldrows[e, 0:16] = earows[e, :]
        pltpu.sync_copy(ldrows.at[pl.ds(0, K), :], lsd_sh.at[dv], add=True)

    tile_base = ci * EPC + si * EPT

    def _loop(c, _):
        _chunk(tile_base + c * CH, CH, dstv)
        return _
    lax.fori_loop(0, NCHUNK, _loop, None)
    if TAIL:
        _chunk(tile_base + NCHUNK * CH, TAIL, dstv_t)

    plsc.subcore_barrier()
    _writeback_slabs(si, lsd_sh, lsd_hbm.at[ci])


@functools.cache
def _sc_pre():
    return pl.kernel(
        _sc_pre_body,
        out_type=jax.ShapeDtypeStruct((NC, N, D), jnp.float32),
        mesh=_mesh(),
        compiler_params=pltpu.CompilerParams(needs_layout_passes=False),
        scratch_types=[
            pltpu.VMEM((CH,), jnp.int32),
            pltpu.VMEM((CH, ED), jnp.float32),
            pltpu.VMEM((CH, D), jnp.float32),
            pltpu.VMEM((TAIL,), jnp.int32),
            pltpu.VMEM_SHARED((N, D), jnp.float32),
        ],
    )


def _sc_edge_body(layer, src_hbm, dst_hbm, ae_hbm, h_hbm, as2d_hbm,
                  p_hbm, dw_hbm,
                  as2v, srcv, dstv, aev, rows, den, srcv_t, dstv_t,
                  p_sh):
    ci = lax.axis_index("c")
    si = lax.axis_index("s")
    z16 = jnp.zeros((16,), jnp.float32)

    for q in range(4):
        pltpu.sync_copy(as2d_hbm.at[pl.ds(q * (N // 2), N // 2)],
                        as2v.at[pl.ds(q * (N // 2), N // 2)])

    def _zrow(r, _):
        for k in range(8):
            rows[r, pl.ds(k * 16, 16)] = z16
        return _
    lax.fori_loop(0, CH, _zrow, None)

    def _zden(r, _):
        den[pl.ds(r * 16, 16)] = z16
        return _
    lax.fori_loop(0, N // 16, _zden, None)
    _zero_slabs(si, rows, p_sh)
    plsc.subcore_barrier()

    def _chunk(base, K, sv, dv):
        pltpu.sync_copy(src_hbm.at[pl.ds(base, K)], sv)
        pltpu.sync_copy(dst_hbm.at[pl.ds(base, K)], dv)
        pltpu.sync_copy(ae_hbm.at[pl.ds(layer * E + base, K)],
                        aev.at[pl.ds(0, K)])
        pltpu.sync_copy(h_hbm.at[sv], rows.at[pl.ds(0, K), :])
        for g in range(K // 16):
            sidx = sv[pl.ds(g * 16, 16)]
            didx = dv[pl.ds(g * 16, 16)]
            a = (plsc.load_gather(as2v, [sidx * 2])
                 + plsc.load_gather(as2v, [didx * 2 + 1])
                 + aev[pl.ds(g * 16, 16)])
            a = jnp.where(a >= 0, a, 0.2 * a)
            w = jnp.exp(a)
            plsc.addupdate_scatter(den, [didx], w)
            for l in range(16):
                c = w[l]
                e = g * 16 + l
                for k in range(8):
                    rows[e, pl.ds(k * 16, 16)] = rows[e, pl.ds(k * 16, 16)] * c
        pltpu.sync_copy(rows.at[pl.ds(0, K), :], p_sh.at[dv], add=True)

    tile_base = ci * EPC + si * EPT

    def _loop(c, _):
        _chunk(tile_base + c * CH, CH, srcv, dstv)
        return _
    lax.fori_loop(0, NCHUNK, _loop, None)
    if TAIL:
        _chunk(tile_base + NCHUNK * CH, TAIL, srcv_t, dstv_t)

    pltpu.sync_copy(den, dw_hbm.at[pl.ds((ci * NS + si) * N, N)])
    plsc.subcore_barrier()
    _writeback_slabs(si, p_sh, p_hbm.at[ci])


@functools.cache
def _make_sc_edge(layer):
    return pl.kernel(
        functools.partial(_sc_edge_body, layer),
        out_type=(jax.ShapeDtypeStruct((NC, N, D), jnp.float32),
                  jax.ShapeDtypeStruct((NW * N,), jnp.float32)),
        mesh=_mesh(),
        compiler_params=pltpu.CompilerParams(needs_layout_passes=False),
        scratch_types=[
            pltpu.VMEM((2 * N,), jnp.float32),
            pltpu.VMEM((CH,), jnp.int32),
            pltpu.VMEM((CH,), jnp.int32),
            pltpu.VMEM((CH,), jnp.float32),
            pltpu.VMEM((CH, D), jnp.float32),
            pltpu.VMEM((N,), jnp.float32),
            pltpu.VMEM((TAIL,), jnp.int32),
            pltpu.VMEM((TAIL,), jnp.int32),
            pltpu.VMEM_SHARED((N, D), jnp.float32),
        ],
    )


def _sc_edge(i):
    return _make_sc_edge(i)


RPT = 320


def _sc_den_body(dwp_hbm, dwr_hbm, acc, tmp, exprows):
    ci = lax.axis_index("c")
    si = lax.axis_index("s")
    w = si * NC + ci
    z16 = jnp.zeros((16,), jnp.float32)
    onehot = jnp.where(lax.iota(jnp.int32, 16) == 0, 1.0, 0.0)

    def _run(row0, nrows):
        def _z(r, _):
            acc[pl.ds(r * 16, 16)] = z16
            return _
        lax.fori_loop(0, nrows // 16, _z, None)
        for w2 in range(NW):
            pltpu.sync_copy(dwp_hbm.at[pl.ds(w2 * N + row0, nrows)],
                            tmp.at[pl.ds(0, nrows)])

            def _add(r, _):
                sl = pl.ds(r * 16, 16)
                acc[sl] = acc[sl] + tmp[sl]
                return _
            lax.fori_loop(0, nrows // 16, _add, None)

        full, rem = divmod(nrows, CH)
        for j in range(full):
            def _expj(g, _):
                v = acc[pl.ds(j * CH + g * 16, 16)]
                for l in range(16):
                    exprows[g * 16 + l, :] = v[l] * onehot
                return _
            lax.fori_loop(0, CH // 16, _expj, None)
            pltpu.sync_copy(exprows,
                            dwr_hbm.at[pl.ds(row0 + j * CH, CH), :])
        if rem:
            def _expr(g, _):
                v = acc[pl.ds(full * CH + g * 16, 16)]
                for l in range(16):
                    exprows[g * 16 + l, :] = v[l] * onehot
                return _
            lax.fori_loop(0, rem // 16, _expr, None)
            pltpu.sync_copy(exprows.at[pl.ds(0, rem), :],
                            dwr_hbm.at[pl.ds(row0 + full * CH, rem), :])

    @pl.when(w < NW - 1)
    def _():
        _run(w * RPT, RPT)

    @pl.when(w == NW - 1)
    def _():
        _run((NW - 1) * RPT, N - (NW - 1) * RPT)


@functools.cache
def _sc_den():
    return pl.kernel(
        _sc_den_body,
        out_type=jax.ShapeDtypeStruct((N, 16), jnp.float32),
        mesh=_mesh(),
        compiler_params=pltpu.CompilerParams(needs_layout_passes=False),
        scratch_types=[
            pltpu.VMEM((RPT,), jnp.float32),
            pltpu.VMEM((RPT,), jnp.float32),
            pltpu.VMEM((CH, 16), jnp.float32),
        ],
    )


_HI = jax.lax.Precision.HIGHEST
NB = 2000
NG = N // NB
EB = 2560
EG = E // EB


def _edge_attn_vectors(we_ref, aee_ref):
    cols = [lax.dot_general(we_ref[l], aee_ref[l][:, None],
                            (((1,), (0,)), ((), ())), precision=_HI)
            for l in range(5)]
    return jnp.concatenate(cols, axis=1)


def _prep_edges_body(ea_ref, we_ref, aee_ref, out_ref):
    V = _edge_attn_vectors(we_ref, aee_ref)
    out_ref[...] = lax.dot_general(V, ea_ref[...], (((0,), (1,)), ((), ())),
                                   precision=_HI)


_prep_edges = pl.pallas_call(
    _prep_edges_body,
    grid=(EG,),
    in_specs=[
        pl.BlockSpec((EB, ED), lambda j: (j, 0)),
        pl.BlockSpec((5, ED, D), lambda j: (0, 0, 0)),
        pl.BlockSpec((5, D), lambda j: (0, 0)),
    ],
    out_specs=pl.BlockSpec((5, EB), lambda j: (0, j)),
    out_shape=jax.ShapeDtypeStruct((5, E), jnp.float32),
)


def _prep_loop_body(lsd_ref, we_ref, aee_ref, out_ref):
    s = lsd_ref[0] + lsd_ref[1]
    la = s[:, 0:16] / jnp.maximum(s[:, 16:17], 1.0)
    V = _edge_attn_vectors(we_ref, aee_ref)
    out_ref[...] = lax.dot_general(la, V, (((1,), (0,)), ((), ())),
                                   precision=_HI)


_prep_loop = pl.pallas_call(
    _prep_loop_body,
    grid=(NG,),
    in_specs=[
        pl.BlockSpec((NC, NB, D), lambda j: (0, j, 0)),
        pl.BlockSpec((5, ED, D), lambda j: (0, 0, 0)),
        pl.BlockSpec((5, D), lambda j: (0, 0)),
    ],
    out_specs=pl.BlockSpec((NB, 5), lambda j: (j, 0)),
    out_shape=jax.ShapeDtypeStruct((N, 5), jnp.float32),
)


def _project(xc, w_ref, a2_ref):
    h = jnp.dot(xc, w_ref[...], precision=_HI)
    as2d = jnp.dot(h, a2_ref[...], precision=_HI)
    return h, as2d


def _pre0_body(x_ref, w_ref, a2_ref, h_ref, as2d_ref):
    h, as2d = _project(x_ref[...], w_ref, a2_ref)
    h_ref[...] = h
    as2d_ref[...] = as2d


_pre0 = pl.pallas_call(
    _pre0_body,
    grid=(NG,),
    in_specs=[
        pl.BlockSpec((NB, D), lambda j: (j, 0)),
        pl.BlockSpec((D, D), lambda j: (0, 0)),
        pl.BlockSpec((D, 2), lambda j: (0, 0)),
    ],
    out_specs=[
        pl.BlockSpec((NB, D), lambda j: (j, 0)),
        pl.BlockSpec((NB, 2), lambda j: (j, 0)),
    ],
    out_shape=[
        jax.ShapeDtypeStruct((N, D), jnp.float32),
        jax.ShapeDtypeStruct((N, 2), jnp.float32),
    ],
)


def _normalize(layer, p_ref, dw_ref, h_ref, as2d_ref, ael_ref, b_ref):
    p = p_ref[0] + p_ref[1]
    den = dw_ref[:, 0:1]
    al = as2d_ref[:, 0:1] + as2d_ref[:, 1:2] + ael_ref[:, layer:layer + 1]
    al = jnp.where(al >= 0, al, 0.2 * al)
    wl = jnp.exp(al)
    return (p + wl * h_ref[...]) / (den + wl + 1e-16) + b_ref[...]


def _post0_body(p_ref, dw_ref, h_ref, as2d_ref, ael_ref, b_ref, w_ref, a2_ref,
                x0_ref, h1_ref, as2d1_ref):
    x0 = _normalize(0, p_ref, dw_ref, h_ref, as2d_ref, ael_ref, b_ref)
    x0_ref[...] = x0
    h, as2d = _project(x0, w_ref, a2_ref)
    h1_ref[...] = h
    as2d1_ref[...] = as2d


_post0 = pl.pallas_call(
    _post0_body,
    grid=(NG,),
    in_specs=[
        pl.BlockSpec((NC, NB, D), lambda j: (0, j, 0)),
        pl.BlockSpec((NB, 16), lambda j: (j, 0)),
        pl.BlockSpec((NB, D), lambda j: (j, 0)),
        pl.BlockSpec((NB, 2), lambda j: (j, 0)),
        pl.BlockSpec((NB, 5), lambda j: (j, 0)),
        pl.BlockSpec((1, D), lambda j: (0, 0)),
        pl.BlockSpec((D, D), lambda j: (0, 0)),
        pl.BlockSpec((D, 2), lambda j: (0, 0)),
    ],
    out_specs=[
        pl.BlockSpec((NB, D), lambda j: (j, 0)),
        pl.BlockSpec((NB, D), lambda j: (j, 0)),
        pl.BlockSpec((NB, 2), lambda j: (j, 0)),
    ],
    out_shape=[
        jax.ShapeDtypeStruct((N, D), jnp.float32),
        jax.ShapeDtypeStruct((N, D), jnp.float32),
        jax.ShapeDtypeStruct((N, 2), jnp.float32),
    ],
)


def _post_mid_body(layer, p_ref, dw_ref, h_ref, as2d_ref, ael_ref, b_ref,
                   x0_ref, mid_ref, w_ref, a2_ref,
                   h1_ref, as2d1_ref):
    cx = _normalize(layer, p_ref, dw_ref, h_ref, as2d_ref, ael_ref, b_ref)
    beta = float(np.log(THETA / layer + 1.0))
    xc = (1.0 - ALPHA) * cx + ALPHA * x0_ref[...]
    xm = jnp.dot(xc, mid_ref[...], precision=_HI)
    xc = (1.0 - beta) * xc + beta * xm
    xc = jnp.where(xc >= 0, xc, 0.01 * xc)
    h, as2d = _project(xc, w_ref, a2_ref)
    h1_ref[...] = h
    as2d1_ref[...] = as2d


def _make_post_mid(layer):
    return pl.pallas_call(
        functools.partial(_post_mid_body, layer),
        grid=(NG,),
        in_specs=[
            pl.BlockSpec((NC, NB, D), lambda j: (0, j, 0)),
            pl.BlockSpec((NB, 16), lambda j: (j, 0)),
            pl.BlockSpec((NB, D), lambda j: (j, 0)),
            pl.BlockSpec((NB, 2), lambda j: (j, 0)),
            pl.BlockSpec((NB, 5), lambda j: (j, 0)),
            pl.BlockSpec((1, D), lambda j: (0, 0)),
            pl.BlockSpec((NB, D), lambda j: (j, 0)),
            pl.BlockSpec((D, D), lambda j: (0, 0)),
            pl.BlockSpec((D, D), lambda j: (0, 0)),
            pl.BlockSpec((D, 2), lambda j: (0, 0)),
        ],
        out_specs=[
            pl.BlockSpec((NB, D), lambda j: (j, 0)),
            pl.BlockSpec((NB, 2), lambda j: (j, 0)),
        ],
        out_shape=[
            jax.ShapeDtypeStruct((N, D), jnp.float32),
            jax.ShapeDtypeStruct((N, 2), jnp.float32),
        ],
    )


_post_mid = {i: _make_post_mid(i) for i in (1, 2, 3)}


def _post_final_body(p_ref, dw_ref, h_ref, as2d_ref, ael_ref, b_ref, out_ref):
    out_ref[...] = _normalize(4, p_ref, dw_ref, h_ref, as2d_ref, ael_ref,
                              b_ref)


_post_final = pl.pallas_call(
    _post_final_body,
    grid=(NG,),
    in_specs=[
        pl.BlockSpec((NC, NB, D), lambda j: (0, j, 0)),
        pl.BlockSpec((NB, 16), lambda j: (j, 0)),
        pl.BlockSpec((NB, D), lambda j: (j, 0)),
        pl.BlockSpec((NB, 2), lambda j: (j, 0)),
        pl.BlockSpec((NB, 5), lambda j: (j, 0)),
        pl.BlockSpec((1, D), lambda j: (0, 0)),
    ],
    out_specs=pl.BlockSpec((NB, D), lambda j: (j, 0)),
    out_shape=jax.ShapeDtypeStruct((N, D), jnp.float32),
)


def kernel(x, edge_index, edge_attr, W, att_src, att_dst, W_edge, att_edge,
           bias, mid_weight):
    src = edge_index[0]
    dst = edge_index[1]
    a2 = [jnp.stack([att_src[i], att_dst[i]], axis=1) for i in range(5)]
    b = [bias[i][None, :] for i in range(5)]

    lsd = _sc_pre()(dst, edge_attr)
    ael = _prep_loop(lsd, W_edge, att_edge)
    ae = _prep_edges(edge_attr, W_edge, att_edge).reshape(5 * E)

    h, as2d = _pre0(x, W[0], a2[0])
    P, Dwp = _sc_edge(0)(src, dst, ae, h, as2d.reshape(2 * N))
    Dw = _sc_den()(Dwp)
    x0, h, as2d = _post0(P, Dw, h, as2d, ael, b[0], W[1], a2[1])
    for i in (1, 2, 3):
        P, Dwp = _sc_edge(i)(src, dst, ae, h, as2d.reshape(2 * N))
        Dw = _sc_den()(Dwp)
        h, as2d = _post_mid[i](P, Dw, h, as2d, ael, b[i], x0,
                               mid_weight[i - 1], W[i + 1], a2[i + 1])
    P, Dwp = _sc_edge(4)(src, dst, ae, h, as2d.reshape(2 * N))
    Dw = _sc_den()(Dwp)
    return _post_final(P, Dw, h, as2d, ael, b[4])
